# Initial kernel scaffold; baseline (speedup 1.0000x reference)
import jax
import jax.numpy as jnp
from jax.experimental import pallas as pl


def kernel(token_ids, E):
    raise NotImplementedError("write your pallas kernel here")


# speedup vs baseline: 1.5589x; 1.5589x over previous
import functools

import jax
import jax.numpy as jnp
from jax import lax
from jax.experimental import pallas as pl
from jax.experimental.pallas import tpu as pltpu
from jax.experimental.pallas import tpu_sc as plsc

NUM_EMBEDDINGS = 1000000
EMBEDDING_DIM = 32

_INFO = plsc.get_sparse_core_info()
_NC, _NS = _INFO.num_cores, _INFO.num_subcores
_NW = _NC * _NS

_B = 16384 * 26
_BPW = _B // _NW
_CHUNK = 1024
_NCHUNK = _BPW // _CHUNK


def _make_kernel():
  mesh = plsc.VectorSubcoreMesh(core_axis_name="c", subcore_axis_name="s")

  @functools.partial(
      pl.kernel,
      out_type=jax.ShapeDtypeStruct((_B, EMBEDDING_DIM), jnp.float32),
      mesh=mesh,
      scratch_types=[
          pltpu.VMEM((_BPW,), jnp.int32),
          pltpu.VMEM((_CHUNK, EMBEDDING_DIM), jnp.float32),
          pltpu.SemaphoreType.DMA,
      ],
      compiler_params=pltpu.CompilerParams(use_tc_tiling_on_sc=False),
  )
  def emb_kernel(idx_hbm, table_hbm, out_hbm, idx_v, rows_v, sem):
    wid = lax.axis_index("s") * _NC + lax.axis_index("c")
    base = wid * _BPW
    pltpu.sync_copy(idx_hbm.at[pl.ds(base, _BPW)], idx_v)
    for c in range(_NCHUNK):
      pltpu.async_copy(
          table_hbm.at[idx_v.at[pl.ds(c * _CHUNK, _CHUNK)]], rows_v, sem
      ).wait()
      pltpu.sync_copy(
          rows_v, out_hbm.at[pl.ds(base + c * _CHUNK, _CHUNK)]
      )

  return emb_kernel


_EMB = _make_kernel()


@jax.jit
def kernel(token_ids, E):
  flat = token_ids.reshape(-1).astype(jnp.int32)
  out = _EMB(flat, E)
  return out.reshape(token_ids.shape + (EMBEDDING_DIM,))


# speedup vs baseline: 1.5772x; 1.0117x over previous
import functools

import jax
import jax.numpy as jnp
from jax import lax
from jax.experimental import pallas as pl
from jax.experimental.pallas import tpu as pltpu
from jax.experimental.pallas import tpu_sc as plsc

NUM_EMBEDDINGS = 1000000
EMBEDDING_DIM = 32

_INFO = plsc.get_sparse_core_info()
_NC, _NS = _INFO.num_cores, _INFO.num_subcores
_NW = _NC * _NS

_B = 16384 * 26
_BPW = _B // _NW
_CHUNK = 832
_NCHUNK = _BPW // _CHUNK
_NBUF = 4
_NGROUP = _NCHUNK // _NBUF


def _make_kernel():
  mesh = plsc.VectorSubcoreMesh(core_axis_name="c", subcore_axis_name="s")

  @functools.partial(
      pl.kernel,
      out_type=jax.ShapeDtypeStruct((_B, EMBEDDING_DIM), jnp.float32),
      mesh=mesh,
      scratch_types=(
          [pltpu.VMEM((_BPW,), jnp.int32)]
          + [pltpu.VMEM((_CHUNK, EMBEDDING_DIM), jnp.float32)] * _NBUF
          + [pltpu.SemaphoreType.DMA] * (2 * _NBUF)
      ),
      compiler_params=pltpu.CompilerParams(use_tc_tiling_on_sc=False),
  )
  def emb_kernel(idx_hbm, table_hbm, out_hbm, idx_v, *scratch):
    rows = scratch[:_NBUF]
    gsem = scratch[_NBUF:2 * _NBUF]
    osem = scratch[2 * _NBUF:]
    wid = lax.axis_index("s") * _NC + lax.axis_index("c")
    base = wid * _BPW
    pltpu.sync_copy(idx_hbm.at[pl.ds(base, _BPW)], idx_v)

    def start_gather(c, b):
      pltpu.async_copy(
          table_hbm.at[idx_v.at[pl.ds(c * _CHUNK, _CHUNK)]], rows[b], gsem[b]
      )

    def wait_gather(b):
      pltpu.make_async_copy(
          table_hbm.at[idx_v.at[pl.ds(0, _CHUNK)]], rows[b], gsem[b]
      ).wait()

    for b in range(_NBUF):
      start_gather(b, b)

    @pl.loop(0, _NGROUP)
    def _group(g):
      for b in range(_NBUF):
        c = g * _NBUF + b
        wait_gather(b)
        pltpu.async_copy(
            rows[b], out_hbm.at[pl.ds(base + c * _CHUNK, _CHUNK)], osem[b]
        )
        pltpu.make_async_copy(
            rows[b], out_hbm.at[pl.ds(base, _CHUNK)], osem[b]
        ).wait()

        @pl.when(g < _NGROUP - 1)
        def _():
          start_gather(c + _NBUF, b)

  return emb_kernel


_EMB = _make_kernel()


@jax.jit
def kernel(token_ids, E):
  flat = token_ids.reshape(-1).astype(jnp.int32)
  out = _EMB(flat, E)
  return out.reshape(token_ids.shape + (EMBEDDING_DIM,))
